# baseline (speedup 1.0000x reference)
import functools
import math

import jax
import jax.numpy as jnp
from jax import lax
from jax.experimental import pallas as pl
from jax.experimental.pallas import tpu as pltpu
from jax.experimental.pallas import tpu_sc as plsc

_B, _S, _V = 4, 2048, 16384
_SMOOTHING = 0.1
_CONF = 1.0 - _SMOOTHING
_R = _B * _S
_FILL = _SMOOTHING / (_V - 1)

_ROW_CONST = (_V - 1) * _FILL * math.log(_FILL) + _CONF * math.log(_CONF)
_TOTAL_CONST = _R * _ROW_CONST

_R_SC = 512
_R_TC = _R - _R_SC

_BLK_ROWS = 256
_NBLK = _R_TC // _BLK_ROWS


def _tc_body(x_ref, tgt_ref, sum_ref, gat_ref):
    i = pl.program_id(0)

    @pl.when(i == 0)
    def _init():
        sum_ref[0, 0] = 0.0
        gat_ref[0, 0] = 0.0

    x = x_ref[...]
    cols = jax.lax.broadcasted_iota(jnp.int32, (_BLK_ROWS, _V), 1)
    mask = cols == tgt_ref[0]
    sum_ref[0, 0] += jnp.sum(x)
    gat_ref[0, 0] += jnp.sum(jnp.where(mask, x, 0.0))


_tc_fused = pl.pallas_call(
    _tc_body,
    grid=(_NBLK,),
    in_specs=[
        pl.BlockSpec((_BLK_ROWS, _V), lambda i: (i, 0)),
        pl.BlockSpec((1, _BLK_ROWS, 1), lambda i: (i, 0, 0)),
    ],
    out_specs=[
        pl.BlockSpec(memory_space=pltpu.SMEM),
        pl.BlockSpec(memory_space=pltpu.SMEM),
    ],
    out_shape=[
        jax.ShapeDtypeStruct((1, 1), jnp.float32),
        jax.ShapeDtypeStruct((1, 1), jnp.float32),
    ],
)

_NC, _NS, _L = 2, 16, 16
_NW = _NC * _NS
_RPW = _R_SC // _NW
_NGG = _RPW // 16
_CW = 2048
_NCC = _V // _CW
_NCHUNK = _NGG * _NCC


def _sc_body(x_hbm, tgt_hbm, out_hbm, tgt_v, buf0, buf1, obuf, acc_v, gacc_v,
             sem0, sem1):
    cid = lax.axis_index("c")
    sid = lax.axis_index("s")
    wid = sid * _NC + cid
    row_base = _R_TC + wid * _RPW

    pltpu.sync_copy(tgt_hbm.at[pl.ds(wid, 1)], tgt_v)

    bufs = (buf0, buf1)
    sems = (sem0, sem1)
    iot = lax.iota(jnp.int32, _L)
    z = jnp.zeros((_L,), jnp.float32)

    def chunk_slice(c):
        gg = c // _NCC
        cc = c % _NCC
        return x_hbm.at[pl.ds(row_base + gg * 16, 16),
                        pl.ds(cc * _CW, _CW)]

    def chunk_sum(buf):
        for r in range(16):
            def step(i, carry, r=r):
                b = i * 256
                v = [buf[r, pl.ds(b + j * _L, _L)] for j in range(16)]
                s01 = (v[0] + v[1]) + (v[2] + v[3])
                s23 = (v[4] + v[5]) + (v[6] + v[7])
                s45 = (v[8] + v[9]) + (v[10] + v[11])
                s67 = (v[12] + v[13]) + (v[14] + v[15])
                acc_v[0, pl.ds(0, _L)] += (s01 + s23) + (s45 + s67)
                return carry

            lax.fori_loop(0, _CW // 256, step, 0)

    acc_v[0, pl.ds(0, _L)] = z
    gacc_v[0, pl.ds(0, _L)] = z

    pltpu.async_copy(chunk_slice(0), buf0, sem0)
    pltpu.async_copy(chunk_slice(1), buf1, sem1)

    @pl.loop(0, _NCHUNK // 2)
    def _outer(o):
        for b in range(2):
            c = o * 2 + b
            buf = bufs[b]
            pltpu.make_async_copy(chunk_slice(c), buf, sems[b]).wait()
            chunk_sum(buf)
            gg = c // _NCC
            cc = c % _NCC
            t16 = tgt_v[0, pl.ds(gg * _L, _L)]
            picks = []
            for r in range(16):
                t_r = t16[r]
                pos = t_r - cc * _CW
                oc = jnp.clip(pos, 0, _CW - 1)
                start = (oc // _L) * _L
                vals = buf[r, pl.ds(start, _L)]
                shift = (pos - start) + (iot - iot)
                picks.append(jnp.where(iot == shift, vals, 0.0))
            p = picks
            for w in (8, 4, 2, 1):
                p = [p[k] + p[k + w] for k in range(w)]
            gacc_v[0, pl.ds(0, _L)] += p[0]

            @pl.when(c + 2 < _NCHUNK)
            def _():
                pltpu.async_copy(chunk_slice(c + 2), buf, sems[b])

    obuf[0, pl.ds(0, _L)] = acc_v[0, pl.ds(0, _L)]
    obuf[0, pl.ds(_L, _L)] = gacc_v[0, pl.ds(0, _L)]
    for k in range(2, 8):
        obuf[0, pl.ds(k * _L, _L)] = z
    pltpu.sync_copy(obuf, out_hbm.at[pl.ds(wid, 1)])


_sc_part = functools.partial(
    pl.kernel,
    mesh=plsc.VectorSubcoreMesh(core_axis_name="c", subcore_axis_name="s"),
    out_type=jax.ShapeDtypeStruct((_NW, 128), jnp.float32),
    scratch_types=[
        pltpu.VMEM((1, 128), jnp.int32),
        pltpu.VMEM((16, _CW), jnp.float32),
        pltpu.VMEM((16, _CW), jnp.float32),
        pltpu.VMEM((1, 128), jnp.float32),
        pltpu.VMEM((1, 128), jnp.float32),
        pltpu.VMEM((1, 128), jnp.float32),
        pltpu.SemaphoreType.DMA,
        pltpu.SemaphoreType.DMA,
    ],
)(_sc_body)


def kernel(x, target):
    x2 = x.reshape(_R, _V)
    tflat = target.reshape(_R).astype(jnp.int32)

    tc_tgt = tflat[:_R_TC].reshape(_NBLK, _BLK_ROWS, 1)
    sc_tgt = jnp.pad(tflat[_R_TC:].reshape(_NW, _RPW),
                     ((0, 0), (0, 128 - _RPW)))

    sums_tc, gats_tc = _tc_fused(x2, tc_tgt)
    parts = _sc_part(x2, sc_tgt)

    sum_x = sums_tc[0, 0] + jnp.sum(parts[:, :_L])
    sum_gather = gats_tc[0, 0] + jnp.sum(parts[:, _L:2 * _L])

    fill = jnp.float32(_FILL)
    conf_m_fill = jnp.float32(_CONF - _FILL)
    return jnp.float32(_TOTAL_CONST) - fill * sum_x - conf_m_fill * sum_gather


# speedup vs baseline: 1.1073x; 1.1073x over previous
import functools
import math

import jax
import jax.numpy as jnp
from jax import lax
from jax.experimental import pallas as pl
from jax.experimental.pallas import tpu as pltpu
from jax.experimental.pallas import tpu_sc as plsc

_B, _S, _V = 4, 2048, 16384
_SMOOTHING = 0.1
_CONF = 1.0 - _SMOOTHING
_R = _B * _S
_FILL = _SMOOTHING / (_V - 1)

_ROW_CONST = (_V - 1) * _FILL * math.log(_FILL) + _CONF * math.log(_CONF)
_TOTAL_CONST = _R * _ROW_CONST

_R_SC = 3072
_R_TC = _R - _R_SC

_BLK_ROWS = 256
_NBLK = _R_TC // _BLK_ROWS


def _tc_body(x_ref, tgt_ref, sum_ref, gat_ref):
    i = pl.program_id(0)

    @pl.when(i == 0)
    def _init():
        sum_ref[0, 0] = 0.0
        gat_ref[0, 0] = 0.0

    x = x_ref[...]
    cols = jax.lax.broadcasted_iota(jnp.int32, (_BLK_ROWS, _V), 1)
    mask = cols == tgt_ref[0]
    sum_ref[0, 0] += jnp.sum(x)
    gat_ref[0, 0] += jnp.sum(jnp.where(mask, x, 0.0))


_tc_fused = pl.pallas_call(
    _tc_body,
    grid=(_NBLK,),
    in_specs=[
        pl.BlockSpec((_BLK_ROWS, _V), lambda i: (i, 0)),
        pl.BlockSpec((1, _BLK_ROWS, 1), lambda i: (i, 0, 0)),
    ],
    out_specs=[
        pl.BlockSpec(memory_space=pltpu.SMEM),
        pl.BlockSpec(memory_space=pltpu.SMEM),
    ],
    out_shape=[
        jax.ShapeDtypeStruct((1, 1), jnp.float32),
        jax.ShapeDtypeStruct((1, 1), jnp.float32),
    ],
)

_NC, _NS, _L = 2, 16, 16
_NW = _NC * _NS
_RPW = _R_SC // _NW
_NGG = _RPW // 16
_CW = 2048
_NCC = _V // _CW
_NCHUNK = _NGG * _NCC


def _sc_body(x_hbm, tgt_hbm, out_hbm, tgt_v, buf0, buf1, obuf, acc_v, gacc_v,
             sem0, sem1):
    cid = lax.axis_index("c")
    sid = lax.axis_index("s")
    wid = sid * _NC + cid
    row_base = _R_TC + wid * _RPW

    pltpu.sync_copy(tgt_hbm.at[pl.ds(wid, 1)], tgt_v)

    bufs = (buf0, buf1)
    sems = (sem0, sem1)
    iot = lax.iota(jnp.int32, _L)
    z = jnp.zeros((_L,), jnp.float32)

    def chunk_slice(c):
        gg = c // _NCC
        cc = c % _NCC
        return x_hbm.at[pl.ds(row_base + gg * 16, 16),
                        pl.ds(cc * _CW, _CW)]

    def chunk_sum(buf):
        for r in range(16):
            def step(i, carry, r=r):
                b = i * 512
                v = [buf[r, pl.ds(b + j * _L, _L)] for j in range(32)]
                p = v
                for w in (16, 8, 4, 2, 1):
                    p = [p[k] + p[k + w] for k in range(w)]
                acc_v[0, pl.ds(0, _L)] += p[0]
                return carry

            lax.fori_loop(0, _CW // 512, step, 0)

    acc_v[0, pl.ds(0, _L)] = z
    gacc_v[0, pl.ds(0, _L)] = z

    pltpu.async_copy(chunk_slice(0), buf0, sem0)
    pltpu.async_copy(chunk_slice(1), buf1, sem1)

    @pl.loop(0, _NCHUNK // 2)
    def _outer(o):
        for b in range(2):
            c = o * 2 + b
            buf = bufs[b]
            pltpu.make_async_copy(chunk_slice(c), buf, sems[b]).wait()
            chunk_sum(buf)
            gg = c // _NCC
            cc = c % _NCC
            t16 = tgt_v[0, pl.ds(gg * _L, _L)]
            picks = []
            for r in range(16):
                t_r = t16[r]
                pos = t_r - cc * _CW
                oc = jnp.clip(pos, 0, _CW - 1)
                start = (oc // _L) * _L
                vals = buf[r, pl.ds(start, _L)]
                shift = (pos - start) + (iot - iot)
                picks.append(jnp.where(iot == shift, vals, 0.0))
            p = picks
            for w in (8, 4, 2, 1):
                p = [p[k] + p[k + w] for k in range(w)]
            gacc_v[0, pl.ds(0, _L)] += p[0]

            @pl.when(c + 2 < _NCHUNK)
            def _():
                pltpu.async_copy(chunk_slice(c + 2), buf, sems[b])

    obuf[0, pl.ds(0, _L)] = acc_v[0, pl.ds(0, _L)]
    obuf[0, pl.ds(_L, _L)] = gacc_v[0, pl.ds(0, _L)]
    for k in range(2, 8):
        obuf[0, pl.ds(k * _L, _L)] = z
    pltpu.sync_copy(obuf, out_hbm.at[pl.ds(wid, 1)])


_sc_part = functools.partial(
    pl.kernel,
    mesh=plsc.VectorSubcoreMesh(core_axis_name="c", subcore_axis_name="s"),
    out_type=jax.ShapeDtypeStruct((_NW, 128), jnp.float32),
    scratch_types=[
        pltpu.VMEM((1, 128), jnp.int32),
        pltpu.VMEM((16, _CW), jnp.float32),
        pltpu.VMEM((16, _CW), jnp.float32),
        pltpu.VMEM((1, 128), jnp.float32),
        pltpu.VMEM((1, 128), jnp.float32),
        pltpu.VMEM((1, 128), jnp.float32),
        pltpu.SemaphoreType.DMA,
        pltpu.SemaphoreType.DMA,
    ],
)(_sc_body)


def kernel(x, target):
    x2 = x.reshape(_R, _V)
    tflat = target.reshape(_R).astype(jnp.int32)

    tc_tgt = tflat[:_R_TC].reshape(_NBLK, _BLK_ROWS, 1)
    sc_tgt = jnp.pad(tflat[_R_TC:].reshape(_NW, _RPW),
                     ((0, 0), (0, 128 - _RPW)))

    sums_tc, gats_tc = _tc_fused(x2, tc_tgt)
    parts = _sc_part(x2, sc_tgt)

    sum_x = sums_tc[0, 0] + jnp.sum(parts[:, :_L])
    sum_gather = gats_tc[0, 0] + jnp.sum(parts[:, _L:2 * _L])

    fill = jnp.float32(_FILL)
    conf_m_fill = jnp.float32(_CONF - _FILL)
    return jnp.float32(_TOTAL_CONST) - fill * sum_x - conf_m_fill * sum_gather
